# speedup vs baseline: 1.2116x; 1.2116x over previous
import functools

import jax
import jax.numpy as jnp
from jax.experimental import pallas as pl
from jax.experimental.pallas import tpu as pltpu

N_NODES = 10000
N_EDGES = 160000
D0 = 128
D1 = 16
HEADS = 3
OUT_CH = 32
HID = HEADS * OUT_CH

POOL_BLK = 8000


def _pool_mlp_body(x_ref, ap_ref, w0_ref, b0_ref, w1_ref, b1_ref, out_ref, acc_ref):
    i = pl.program_id(0)
    n = pl.num_programs(0)
    xb = x_ref[...]
    ap = ap_ref[...]
    s = jax.nn.sigmoid(jnp.sum(xb * ap, axis=1, keepdims=True))
    g = xb * s
    m = jnp.max(g, axis=0)

    @pl.when(i == 0)
    def _init():
        acc_ref[...] = jnp.full_like(acc_ref[...], -jnp.inf)

    acc_ref[0, :HID] = jnp.maximum(acc_ref[0, :HID], m)

    @pl.when(i == n - 1)
    def _final():
        y = acc_ref[0:1, :HID]
        h = jax.nn.relu(
            jnp.dot(y, w0_ref[...], preferred_element_type=jnp.float32)
            + b0_ref[...]
        )
        o = jax.nn.relu(jnp.sum(h * w1_ref[...], axis=1, keepdims=True) + b1_ref[...])
        out_ref[...] = o


def _pool_mlp(x1, att_pool, W0, b0, W1, b1):
    n = x1.shape[0]
    grid = n // POOL_BLK
    out = pl.pallas_call(
        _pool_mlp_body,
        grid=(grid,),
        in_specs=[
            pl.BlockSpec((POOL_BLK, HID), lambda i: (i, 0)),
            pl.BlockSpec((1, HID), lambda i: (0, 0)),
            pl.BlockSpec((HID, 64), lambda i: (0, 0)),
            pl.BlockSpec((1, 64), lambda i: (0, 0)),
            pl.BlockSpec((1, 64), lambda i: (0, 0)),
            pl.BlockSpec((1, 1), lambda i: (0, 0)),
        ],
        out_specs=pl.BlockSpec((1, 1), lambda i: (0, 0)),
        out_shape=jax.ShapeDtypeStruct((1, 1), jnp.float32),
        scratch_shapes=[pltpu.VMEM((8, 128), jnp.float32)],
    )(x1, att_pool.reshape(1, HID), W0, b0.reshape(1, 64), W1.reshape(1, 64),
      b1.reshape(1, 1))
    return out.reshape(1)


def _leaky_relu(x):
    return jnp.where(x >= 0, x, 0.2 * x)


def _mha(x, idx, W, a_src, a_dst):
    n = x.shape[0]
    xm = (x @ W).reshape(n, HEADS, OUT_CH)
    tgt = idx[0]
    src = idx[1]
    a_s = (xm * a_src[None, :, :]).sum(-1)
    a_d = (xm * a_dst[None, :, :]).sum(-1)
    logits = _leaky_relu(a_s[src] + a_d[tgt])
    gmax = jnp.max(logits)
    e = jnp.exp(logits - gmax)
    den = jax.ops.segment_sum(e, tgt, num_segments=n)
    num = jax.ops.segment_sum(xm[src] * e[:, :, None], tgt, num_segments=n)
    agg = num / (den[:, :, None] + 1e-16)
    return agg.reshape(n, HID)


def _can_layer(x, low, up, Wl, als, ald, Wu, aus, aud, Ws):
    lx = _mha(x, low, Wl, als, ald)
    ux = _mha(x, up, Wu, aus, aud)
    wx = (x @ Ws) * (1.0 + 1e-6)
    return jax.nn.relu(lx + ux + wx)


def kernel(x_0, x_1, neighborhood_0_to_0, lower_neighborhood, upper_neighborhood,
           edge_indices, lift_att, Wl1, als1, ald1, Wu1, aus1, aud1, Ws1,
           Wl2, als2, ald2, Wu2, aus2, aud2, Ws2, att_pool, W0, b0, W1, b1):
    src = neighborhood_0_to_0[0]
    tgt = neighborhood_0_to_0[1]
    lifted = jax.nn.relu(x_0[src] @ lift_att[:D0] + x_0[tgt] @ lift_att[D0:])
    x1 = jnp.concatenate([lifted, x_1], axis=1)
    x1 = _can_layer(x1, lower_neighborhood, upper_neighborhood,
                    Wl1, als1, ald1, Wu1, aus1, aud1, Ws1)
    x1 = _can_layer(x1, lower_neighborhood, upper_neighborhood,
                    Wl2, als2, ald2, Wu2, aus2, aud2, Ws2)
    return _pool_mlp(x1, att_pool, W0, b0, W1, b1)


# speedup vs baseline: 44.2228x; 36.5006x over previous
import functools

import jax
import jax.numpy as jnp
from jax import lax
from jax.experimental import pallas as pl
from jax.experimental.pallas import tpu as pltpu
from jax.experimental.pallas import tpu_sc as plsc

N_NODES = 10000
N_EDGES = 160000
E_NB = 640000
D0 = 128
D1 = 16
HEADS = 3
OUT_CH = 32
HID = HEADS * OUT_CH

NRANGE = 16
RNG = N_EDGES // NRANGE
RPT = RNG // 16
NBLK = 25
EPT = E_NB // 16
ECH = 800
NCH = EPT // ECH
K = 128
SELCAP = ECH + K + 16
AW = 112

_mesh = plsc.VectorSubcoreMesh(core_axis_name="c", subcore_axis_name="s")


def _iota16():
    return jnp.arange(16, dtype=jnp.int32)


def _full16(v):
    return jnp.full((16,), v, dtype=jnp.int32)


def _leaky_v(x):
    return jnp.where(x >= 0, x, 0.2 * x)


@functools.partial(
    pl.kernel,
    out_type=jax.ShapeDtypeStruct((N_EDGES, HID), jnp.float32),
    mesh=_mesh,
    compiler_params=pltpu.CompilerParams(use_tc_tiling_on_sc=False,
                                         needs_layout_passes=False),
    scratch_types=dict(
        selt=pltpu.VMEM((SELCAP,), jnp.int32),
        selsrc=pltpu.VMEM((SELCAP,), jnp.int32),
        tch=pltpu.VMEM((ECH,), jnp.int32),
        sch=pltpu.VMEM((ECH,), jnp.int32),
        gsrc=pltpu.VMEM((K,), jnp.int32),
        gtgt=pltpu.VMEM((K,), jnp.int32),
        lidx=pltpu.VMEM((K,), jnp.int32),
        asd_s=pltpu.VMEM((K, 16), jnp.float32),
        asd_t=pltpu.VMEM((K, 16), jnp.float32),
        xmb=pltpu.VMEM((K, HID), jnp.float32),
        ebuf=pltpu.VMEM((4 * K,), jnp.float32),
        msgs=pltpu.VMEM((K, AW), jnp.float32),
        nbuf=pltpu.VMEM((NBLK, AW), jnp.float32),
        obuf=pltpu.VMEM((NBLK, HID), jnp.float32),
        mvbuf=pltpu.VMEM((16,), jnp.float32),
        acc=pltpu.VMEM_SHARED((RNG, AW), jnp.float32),
    ),
)
def _sc_mha(asd_hbm, xm_hbm, src_hbm, tgt_hbm, maxv_hbm, z2d_hbm, z1d_hbm,
            out_hbm,
            selt, selsrc, tch, sch, gsrc, gtgt, lidx,
            asd_s, asd_t, xmb, ebuf, msgs, nbuf, obuf, mvbuf, acc):
    cid = lax.axis_index("c")
    tid = lax.axis_index("s")
    iota = _iota16()

    pltpu.sync_copy(z2d_hbm, nbuf)
    pltpu.sync_copy(z1d_hbm, ebuf)
    pltpu.sync_copy(maxv_hbm, mvbuf)
    cvec = mvbuf[...]

    zi = jnp.zeros((16,), jnp.int32)

    def _zs(i, _):
        selt[pl.ds(i * 16, 16)] = zi
        selsrc[pl.ds(i * 16, 16)] = zi
        return 0
    lax.fori_loop(0, SELCAP // 16, _zs, 0)

    def _flush_group(base, nsel, lo):
        def _mkidx(j, _c2):
            lt = selt[pl.ds(base + j * 16, 16)]
            ls = selsrc[pl.ds(base + j * 16, 16)]
            lidx[pl.ds(j * 16, 16)] = lt
            gtgt[pl.ds(j * 16, 16)] = lt + lo
            gsrc[pl.ds(j * 16, 16)] = ls
            return 0
        lax.fori_loop(0, K // 16, _mkidx, 0)

        pltpu.sync_copy(asd_hbm.at[gsrc], asd_s)
        pltpu.sync_copy(asd_hbm.at[gtgt], asd_t)
        pltpu.sync_copy(xm_hbm.at[gsrc], xmb)

        def _egrp(j, _c3):
            rows = iota + j * 16
            valid = (base + rows) < nsel
            for h in range(HEADS):
                a1 = plsc.load_gather(asd_s, [rows, _full16(h)])
                a2 = plsc.load_gather(asd_t, [rows, _full16(3 + h)])
                e = jnp.exp(_leaky_v(a1 + a2) - cvec)
                e = jnp.where(valid, e, 0.0)
                ebuf[pl.ds(h * K + j * 16, 16)] = e
            return 0
        lax.fori_loop(0, K // 16, _egrp, 0)

        def _mrow(i, _c4):
            ev = plsc.load_gather(ebuf, [jnp.minimum(iota, 3) * K + i])
            msgs[i, pl.ds(HID, 16)] = ev
            for h in range(HEADS):
                eh = plsc.load_gather(ebuf, [_full16(h * K) + i])
                for b in range(2):
                    c0 = h * OUT_CH + b * 16
                    xv = xmb[i, pl.ds(c0, 16)]
                    msgs[i, pl.ds(c0, 16)] = xv * eh
            return 0
        lax.fori_loop(0, K, _mrow, 0)

        pltpu.sync_copy(msgs, acc.at[lidx], add=True)

    def _range_body(ri, _):
        p = cid + 2 * ri
        lo = p * RNG

        def _zacc(z, _c):
            pltpu.sync_copy(nbuf, acc.at[pl.ds(tid * RPT + z * NBLK, NBLK)])
            return 0
        lax.fori_loop(0, RPT // NBLK, _zacc, 0)
        plsc.subcore_barrier()

        def _chunk(ch, nsel):
            eoff = tid * EPT + ch * ECH
            pltpu.sync_copy(tgt_hbm.at[pl.ds(eoff, ECH)], tch)
            pltpu.sync_copy(src_hbm.at[pl.ds(eoff, ECH)], sch)

            def _vg(j, ns):
                t = tch[pl.ds(j * 16, 16)]
                s = sch[pl.ds(j * 16, 16)]
                inb = (t >= lo) & (t < lo + RNG)
                plsc.store_compressed(selt.at[pl.ds(ns, 16)], t - lo, mask=inb)
                plsc.store_compressed(selsrc.at[pl.ds(ns, 16)], s, mask=inb)
                cnt = jnp.max(plsc.all_reduce_population_count(inb))
                return ns + cnt
            nsel = lax.fori_loop(0, ECH // 16, _vg, nsel)

            ngr = nsel // K

            def _fl(g, _c):
                _flush_group(g * K, nsel, lo)
                return 0
            lax.fori_loop(0, ngr, _fl, 0)

            rem = nsel - ngr * K

            def _mv(j, _c):
                vt = selt[pl.ds(ngr * K + j * 16, 16)]
                vs = selsrc[pl.ds(ngr * K + j * 16, 16)]
                selt[pl.ds(j * 16, 16)] = vt
                selsrc[pl.ds(j * 16, 16)] = vs
                return 0
            lax.fori_loop(0, K // 16, _mv, 0)
            return rem
        nsel = lax.fori_loop(0, NCH, _chunk, jnp.int32(0))

        ngroups = (nsel + (K - 1)) // K

        def _flast(g, _c):
            _flush_group(g * K, nsel, lo)
            return 0
        lax.fori_loop(0, ngroups, _flast, 0)
        plsc.subcore_barrier()

        def _nblk(b, _c):
            row0 = tid * RPT + b * NBLK
            pltpu.sync_copy(acc.at[pl.ds(row0, NBLK)], nbuf)

            def _nrow(r, _c2):
                fr = _full16(r)
                for h in range(HEADS):
                    d = plsc.load_gather(nbuf, [fr, _full16(HID + h)])
                    d = d + 1e-16
                    for bb in range(2):
                        c0 = h * OUT_CH + bb * 16
                        u = nbuf[r, pl.ds(c0, 16)]
                        obuf[r, pl.ds(c0, 16)] = u / d
                return 0
            lax.fori_loop(0, NBLK, _nrow, 0)
            pltpu.sync_copy(obuf, out_hbm.at[pl.ds(lo + row0, NBLK)])
            return 0
        lax.fori_loop(0, RPT // NBLK, _nblk, 0)

        pltpu.sync_copy(z2d_hbm, nbuf)
        plsc.subcore_barrier()
        return 0

    lax.fori_loop(0, NRANGE // 2, _range_body, 0)


POOL_BLK = 8000


def _pool_mlp_body(x_ref, ap_ref, w0_ref, b0_ref, w1_ref, b1_ref, out_ref, acc_ref):
    i = pl.program_id(0)
    n = pl.num_programs(0)
    xb = x_ref[...]
    ap = ap_ref[...]
    s = jax.nn.sigmoid(jnp.sum(xb * ap, axis=1, keepdims=True))
    g = xb * s
    m = jnp.max(g, axis=0)

    @pl.when(i == 0)
    def _init():
        acc_ref[...] = jnp.full_like(acc_ref[...], -jnp.inf)

    acc_ref[0, :HID] = jnp.maximum(acc_ref[0, :HID], m)

    @pl.when(i == n - 1)
    def _final():
        y = acc_ref[0:1, :HID]
        h = jax.nn.relu(
            jnp.dot(y, w0_ref[...], preferred_element_type=jnp.float32)
            + b0_ref[...]
        )
        o = jax.nn.relu(jnp.sum(h * w1_ref[...], axis=1, keepdims=True) + b1_ref[...])
        out_ref[...] = o


def _pool_mlp(x1, att_pool, W0, b0, W1, b1):
    grid = x1.shape[0] // POOL_BLK
    out = pl.pallas_call(
        _pool_mlp_body,
        grid=(grid,),
        in_specs=[
            pl.BlockSpec((POOL_BLK, HID), lambda i: (i, 0)),
            pl.BlockSpec((1, HID), lambda i: (0, 0)),
            pl.BlockSpec((HID, 64), lambda i: (0, 0)),
            pl.BlockSpec((1, 64), lambda i: (0, 0)),
            pl.BlockSpec((1, 64), lambda i: (0, 0)),
            pl.BlockSpec((1, 1), lambda i: (0, 0)),
        ],
        out_specs=pl.BlockSpec((1, 1), lambda i: (0, 0)),
        out_shape=jax.ShapeDtypeStruct((1, 1), jnp.float32),
        scratch_shapes=[pltpu.VMEM((8, 128), jnp.float32)],
    )(x1, att_pool.reshape(1, HID), W0, b0.reshape(1, 64), W1.reshape(1, 64),
      b1.reshape(1, 1))
    return out.reshape(1)


def _mha(x, idx, W, a_src, a_dst):
    xm = x @ W
    cmat = jnp.zeros((HID, 16), jnp.float32)
    for h in range(HEADS):
        cmat = cmat.at[h * OUT_CH:(h + 1) * OUT_CH, h].set(a_src[h])
        cmat = cmat.at[h * OUT_CH:(h + 1) * OUT_CH, 3 + h].set(a_dst[h])
    asd = jnp.dot(xm, cmat, precision=jax.lax.Precision.HIGHEST)
    mm = jnp.max(asd, axis=0)
    c = _leaky_v(jnp.max(mm[:HEADS] + mm[HEADS:2 * HEADS]))
    maxv = jnp.full((16,), c, jnp.float32)
    tgt = idx[0]
    src = idx[1]
    z2d = jnp.zeros((NBLK, AW), jnp.float32)
    z1d = jnp.zeros((4 * K,), jnp.float32)
    return _sc_mha(asd, xm, src, tgt, maxv, z2d, z1d)


def _can_layer(x, low, up, Wl, als, ald, Wu, aus, aud, Ws):
    lx = _mha(x, low, Wl, als, ald)
    ux = _mha(x, up, Wu, aus, aud)
    wx = (x @ Ws) * (1.0 + 1e-6)
    return jax.nn.relu(lx + ux + wx)


def kernel(x_0, x_1, neighborhood_0_to_0, lower_neighborhood, upper_neighborhood,
           edge_indices, lift_att, Wl1, als1, ald1, Wu1, aus1, aud1, Ws1,
           Wl2, als2, ald2, Wu2, aus2, aud2, Ws2, att_pool, W0, b0, W1, b1):
    src = neighborhood_0_to_0[0]
    tgt = neighborhood_0_to_0[1]
    lifted = jax.nn.relu(x_0[src] @ lift_att[:D0] + x_0[tgt] @ lift_att[D0:])
    x1 = jnp.concatenate([lifted, x_1], axis=1)
    x1 = _can_layer(x1, lower_neighborhood, upper_neighborhood,
                    Wl1, als1, ald1, Wu1, aus1, aud1, Ws1)
    x1 = _can_layer(x1, lower_neighborhood, upper_neighborhood,
                    Wl2, als2, ald2, Wu2, aus2, aud2, Ws2)
    return _pool_mlp(x1, att_pool, W0, b0, W1, b1)


# speedup vs baseline: 46.9037x; 1.0606x over previous
import functools

import jax
import jax.numpy as jnp
from jax import lax
from jax.experimental import pallas as pl
from jax.experimental.pallas import tpu as pltpu
from jax.experimental.pallas import tpu_sc as plsc

N_NODES = 10000
N_EDGES = 160000
E_NB = 640000
D0 = 128
D1 = 16
HEADS = 3
OUT_CH = 32
HID = HEADS * OUT_CH

NRANGE = 16
RNG = N_EDGES // NRANGE
RPT = RNG // 16
NBLK = 25
EPT = E_NB // 16
ECH = 800
NCH = EPT // ECH
K = 128
SELCAP = ECH + K + 16
AW = 112

_mesh = plsc.VectorSubcoreMesh(core_axis_name="c", subcore_axis_name="s")


def _iota16():
    return jnp.arange(16, dtype=jnp.int32)


def _full16(v):
    return jnp.full((16,), v, dtype=jnp.int32)


def _leaky_v(x):
    return jnp.where(x >= 0, x, 0.2 * x)


@functools.partial(
    pl.kernel,
    out_type=jax.ShapeDtypeStruct((N_EDGES, HID), jnp.float32),
    mesh=_mesh,
    compiler_params=pltpu.CompilerParams(use_tc_tiling_on_sc=False,
                                         needs_layout_passes=False),
    scratch_types=dict(
        selt=pltpu.VMEM((SELCAP,), jnp.int32),
        selsrc=pltpu.VMEM((SELCAP,), jnp.int32),
        tch=pltpu.VMEM((ECH,), jnp.int32),
        sch=pltpu.VMEM((ECH,), jnp.int32),
        gsrc=pltpu.VMEM((K,), jnp.int32),
        gtgt=pltpu.VMEM((K,), jnp.int32),
        lidx=pltpu.VMEM((K,), jnp.int32),
        asd_s=pltpu.VMEM((K, 16), jnp.float32),
        asd_t=pltpu.VMEM((K, 16), jnp.float32),
        xmb=pltpu.VMEM((K, HID), jnp.float32),
        ebuf=pltpu.VMEM((4 * K,), jnp.float32),
        msgs=pltpu.VMEM((K, AW), jnp.float32),
        nbuf=pltpu.VMEM((NBLK, AW), jnp.float32),
        obuf=pltpu.VMEM((NBLK, HID), jnp.float32),
        mvbuf=pltpu.VMEM((16,), jnp.float32),
        acc=pltpu.VMEM_SHARED((RNG, AW), jnp.float32),
    ),
)
def _sc_mha(asd_hbm, xm_hbm, src_hbm, tgt_hbm, maxv_hbm, z2d_hbm, z1d_hbm,
            out_hbm,
            selt, selsrc, tch, sch, gsrc, gtgt, lidx,
            asd_s, asd_t, xmb, ebuf, msgs, nbuf, obuf, mvbuf, acc):
    cid = lax.axis_index("c")
    tid = lax.axis_index("s")
    iota = _iota16()

    pltpu.sync_copy(z2d_hbm, nbuf)
    pltpu.sync_copy(z1d_hbm, ebuf)
    pltpu.sync_copy(maxv_hbm, mvbuf)
    cvec = mvbuf[...]

    zi = jnp.zeros((16,), jnp.int32)

    def _zs(i, _):
        selt[pl.ds(i * 16, 16)] = zi
        selsrc[pl.ds(i * 16, 16)] = zi
        return 0
    lax.fori_loop(0, SELCAP // 16, _zs, 0)

    def _flush_group(base, nsel, lo):
        def _mkidx(j, _c2):
            lt = selt[pl.ds(base + j * 16, 16)]
            ls = selsrc[pl.ds(base + j * 16, 16)]
            lidx[pl.ds(j * 16, 16)] = lt
            gtgt[pl.ds(j * 16, 16)] = lt + lo
            gsrc[pl.ds(j * 16, 16)] = ls
            return 0
        lax.fori_loop(0, K // 16, _mkidx, 0)

        pltpu.sync_copy(asd_hbm.at[gsrc], asd_s)
        pltpu.sync_copy(asd_hbm.at[gtgt], asd_t)
        pltpu.sync_copy(xm_hbm.at[gsrc], xmb)

        def _egrp(j, _c3):
            rows = iota + j * 16
            valid = (base + rows) < nsel
            for h in range(HEADS):
                a1 = plsc.load_gather(asd_s, [rows, _full16(h)])
                a2 = plsc.load_gather(asd_t, [rows, _full16(3 + h)])
                e = jnp.exp(_leaky_v(a1 + a2) - cvec)
                e = jnp.where(valid, e, 0.0)
                ebuf[pl.ds(h * K + j * 16, 16)] = e
            return 0
        lax.fori_loop(0, K // 16, _egrp, 0)

        def _mrow(i, _c4):
            ev = plsc.load_gather(ebuf, [jnp.minimum(iota, 3) * K + i])
            msgs[i, pl.ds(HID, 16)] = ev
            for h in range(HEADS):
                eh = plsc.load_gather(ebuf, [_full16(h * K) + i])
                for b in range(2):
                    c0 = h * OUT_CH + b * 16
                    xv = xmb[i, pl.ds(c0, 16)]
                    msgs[i, pl.ds(c0, 16)] = xv * eh
            return 0
        lax.fori_loop(0, K, _mrow, 0)

        pltpu.sync_copy(msgs, acc.at[lidx], add=True)

    def _range_body(ri, _):
        p = cid + 2 * ri
        lo = p * RNG

        def _zacc(z, _c):
            pltpu.sync_copy(nbuf, acc.at[pl.ds(tid * RPT + z * NBLK, NBLK)])
            return 0
        lax.fori_loop(0, RPT // NBLK, _zacc, 0)
        plsc.subcore_barrier()

        def _chunk(ch, nsel):
            eoff = tid * EPT + ch * ECH
            pltpu.sync_copy(tgt_hbm.at[pl.ds(eoff, ECH)], tch)
            pltpu.sync_copy(src_hbm.at[pl.ds(eoff, ECH)], sch)

            def _vg(j, ns):
                t = tch[pl.ds(j * 16, 16)]
                s = sch[pl.ds(j * 16, 16)]
                inb = (t >= lo) & (t < lo + RNG)
                plsc.store_compressed(selt.at[pl.ds(ns, 16)], t - lo, mask=inb)
                plsc.store_compressed(selsrc.at[pl.ds(ns, 16)], s, mask=inb)
                cnt = jnp.max(plsc.all_reduce_population_count(inb))
                return ns + cnt
            nsel = lax.fori_loop(0, ECH // 16, _vg, nsel)

            ngr = nsel // K

            def _fl(g, _c):
                _flush_group(g * K, nsel, lo)
                return 0
            lax.fori_loop(0, ngr, _fl, 0)

            rem = nsel - ngr * K

            def _mv(j, _c):
                vt = selt[pl.ds(ngr * K + j * 16, 16)]
                vs = selsrc[pl.ds(ngr * K + j * 16, 16)]
                selt[pl.ds(j * 16, 16)] = vt
                selsrc[pl.ds(j * 16, 16)] = vs
                return 0
            lax.fori_loop(0, K // 16, _mv, 0)
            return rem
        nsel = lax.fori_loop(0, NCH, _chunk, jnp.int32(0))

        ngroups = (nsel + (K - 1)) // K

        def _flast(g, _c):
            _flush_group(g * K, nsel, lo)
            return 0
        lax.fori_loop(0, ngroups, _flast, 0)
        plsc.subcore_barrier()

        def _nblk(b, _c):
            row0 = tid * RPT + b * NBLK
            pltpu.sync_copy(acc.at[pl.ds(row0, NBLK)], nbuf)

            def _nrow(r, _c2):
                fr = _full16(r)
                for h in range(HEADS):
                    d = plsc.load_gather(nbuf, [fr, _full16(HID + h)])
                    d = d + 1e-16
                    for bb in range(2):
                        c0 = h * OUT_CH + bb * 16
                        u = nbuf[r, pl.ds(c0, 16)]
                        obuf[r, pl.ds(c0, 16)] = u / d
                return 0
            lax.fori_loop(0, NBLK, _nrow, 0)
            pltpu.sync_copy(obuf, out_hbm.at[pl.ds(lo + row0, NBLK)])
            return 0
        lax.fori_loop(0, RPT // NBLK, _nblk, 0)

        pltpu.sync_copy(z2d_hbm, nbuf)
        plsc.subcore_barrier()
        return 0

    lax.fori_loop(0, NRANGE // 2, _range_body, 0)


LCH = 200
LPT = N_EDGES // 32


@functools.partial(
    pl.kernel,
    out_type=jax.ShapeDtypeStruct((N_EDGES, D0), jnp.float32),
    mesh=_mesh,
    compiler_params=pltpu.CompilerParams(use_tc_tiling_on_sc=False,
                                         needs_layout_passes=False),
    scratch_types=dict(
        sidx=pltpu.VMEM((LCH,), jnp.int32),
        tidx=pltpu.VMEM((LCH,), jnp.int32),
        y1b=pltpu.VMEM((LCH, D0), jnp.float32),
        y2b=pltpu.VMEM((LCH, D0), jnp.float32),
        ob=pltpu.VMEM((LCH, D0), jnp.float32),
    ),
)
def _sc_lift(y1_hbm, y2_hbm, s_hbm, t_hbm, out_hbm, sidx, tidx, y1b, y2b, ob):
    cid = lax.axis_index("c")
    tid = lax.axis_index("s")
    wid = tid * 2 + cid
    base = wid * LPT

    def _chunk(ch, _):
        eoff = base + ch * LCH
        pltpu.sync_copy(s_hbm.at[pl.ds(eoff, LCH)], sidx)
        pltpu.sync_copy(t_hbm.at[pl.ds(eoff, LCH)], tidx)
        pltpu.sync_copy(y1_hbm.at[sidx], y1b)
        pltpu.sync_copy(y2_hbm.at[tidx], y2b)

        def _row(r, _c):
            for c8 in range(D0 // 16):
                v = y1b[r, pl.ds(c8 * 16, 16)] + y2b[r, pl.ds(c8 * 16, 16)]
                ob[r, pl.ds(c8 * 16, 16)] = jnp.maximum(v, 0.0)
            return 0
        lax.fori_loop(0, LCH, _row, 0)
        pltpu.sync_copy(ob, out_hbm.at[pl.ds(eoff, LCH)])
        return 0
    lax.fori_loop(0, LPT // LCH, _chunk, 0)


PRE_BLK = 4000
_HI = jax.lax.Precision.HIGHEST


def _lift_proj_body(x0_ref, a1_ref, a2_ref, y1_ref, y2_ref):
    x = x0_ref[...]
    y1_ref[...] = jnp.dot(x, a1_ref[...], preferred_element_type=jnp.float32)
    y2_ref[...] = jnp.dot(x, a2_ref[...], preferred_element_type=jnp.float32)


def _lift_proj(x_0, a1, a2):
    return pl.pallas_call(
        _lift_proj_body,
        grid=(5,),
        in_specs=[
            pl.BlockSpec((N_NODES // 5, D0), lambda i: (i, 0)),
            pl.BlockSpec((D0, D0), lambda i: (0, 0)),
            pl.BlockSpec((D0, D0), lambda i: (0, 0)),
        ],
        out_specs=[
            pl.BlockSpec((N_NODES // 5, D0), lambda i: (i, 0)),
            pl.BlockSpec((N_NODES // 5, D0), lambda i: (i, 0)),
        ],
        out_shape=[jax.ShapeDtypeStruct((N_NODES, D0), jnp.float32)] * 2,
    )(x_0, a1, a2)


def _pre_maxupdate(i, n, asdl, asdu, maxes_ref, mxs_ref):
    ml = jnp.max(asdl, axis=0).reshape(1, 16)
    mu = jnp.max(asdu, axis=0).reshape(1, 16)

    @pl.when(i == 0)
    def _init():
        mxs_ref[...] = jnp.full_like(mxs_ref[...], -jnp.inf)

    mxs_ref[0:1, :16] = jnp.maximum(mxs_ref[0:1, :16], ml)
    mxs_ref[1:2, :16] = jnp.maximum(mxs_ref[1:2, :16], mu)

    @pl.when(i == n - 1)
    def _final():
        maxes_ref[...] = mxs_ref[0:2, :16]


def _pre1_body(xa_ref, xb_ref, wla_ref, wlb_ref, wua_ref, wub_ref,
               wsa_ref, wsb_ref, cl_ref, cu_ref,
               xml_ref, xmu_ref, wx_ref, asdl_ref, asdu_ref, maxes_ref,
               mxs_ref):
    i = pl.program_id(0)
    n = pl.num_programs(0)
    xa = xa_ref[...]
    xb = xb_ref[...]
    xml = (jnp.dot(xa, wla_ref[...], preferred_element_type=jnp.float32)
           + jnp.dot(xb, wlb_ref[...], preferred_element_type=jnp.float32))
    xmu = (jnp.dot(xa, wua_ref[...], preferred_element_type=jnp.float32)
           + jnp.dot(xb, wub_ref[...], preferred_element_type=jnp.float32))
    wx = (jnp.dot(xa, wsa_ref[...], preferred_element_type=jnp.float32)
          + jnp.dot(xb, wsb_ref[...], preferred_element_type=jnp.float32))
    xml_ref[...] = xml
    xmu_ref[...] = xmu
    wx_ref[...] = wx * (1.0 + 1e-6)
    asdl = jnp.dot(xml, cl_ref[...], precision=_HI,
                   preferred_element_type=jnp.float32)
    asdu = jnp.dot(xmu, cu_ref[...], precision=_HI,
                   preferred_element_type=jnp.float32)
    asdl_ref[...] = asdl
    asdu_ref[...] = asdu
    _pre_maxupdate(i, n, asdl, asdu, maxes_ref, mxs_ref)


def _pre1(lifted, x_1, Wl, Wu, Ws, cl, cu):
    grid = N_EDGES // PRE_BLK
    outs = pl.pallas_call(
        _pre1_body,
        grid=(grid,),
        in_specs=[
            pl.BlockSpec((PRE_BLK, D0), lambda i: (i, 0)),
            pl.BlockSpec((PRE_BLK, D1), lambda i: (i, 0)),
        ] + [pl.BlockSpec((D0, HID), lambda i: (0, 0)),
             pl.BlockSpec((D1, HID), lambda i: (0, 0))] * 3
        + [pl.BlockSpec((HID, 16), lambda i: (0, 0))] * 2,
        out_specs=[
            pl.BlockSpec((PRE_BLK, HID), lambda i: (i, 0)),
            pl.BlockSpec((PRE_BLK, HID), lambda i: (i, 0)),
            pl.BlockSpec((PRE_BLK, HID), lambda i: (i, 0)),
            pl.BlockSpec((PRE_BLK, 16), lambda i: (i, 0)),
            pl.BlockSpec((PRE_BLK, 16), lambda i: (i, 0)),
            pl.BlockSpec((2, 16), lambda i: (0, 0)),
        ],
        out_shape=[
            jax.ShapeDtypeStruct((N_EDGES, HID), jnp.float32),
            jax.ShapeDtypeStruct((N_EDGES, HID), jnp.float32),
            jax.ShapeDtypeStruct((N_EDGES, HID), jnp.float32),
            jax.ShapeDtypeStruct((N_EDGES, 16), jnp.float32),
            jax.ShapeDtypeStruct((N_EDGES, 16), jnp.float32),
            jax.ShapeDtypeStruct((2, 16), jnp.float32),
        ],
        scratch_shapes=[pltpu.VMEM((8, 128), jnp.float32)],
    )(lifted, x_1, Wl[:D0], Wl[D0:], Wu[:D0], Wu[D0:], Ws[:D0], Ws[D0:],
      cl, cu)
    return outs


def _pre2_body(al_ref, au_ref, wxp_ref, wl_ref, wu_ref, ws_ref, cl_ref, cu_ref,
               xml_ref, xmu_ref, wx_ref, asdl_ref, asdu_ref, maxes_ref,
               mxs_ref):
    i = pl.program_id(0)
    n = pl.num_programs(0)
    x = jax.nn.relu(al_ref[...] + au_ref[...] + wxp_ref[...])
    xml = jnp.dot(x, wl_ref[...], preferred_element_type=jnp.float32)
    xmu = jnp.dot(x, wu_ref[...], preferred_element_type=jnp.float32)
    xml_ref[...] = xml
    xmu_ref[...] = xmu
    wx_ref[...] = jnp.dot(x, ws_ref[...],
                          preferred_element_type=jnp.float32) * (1.0 + 1e-6)
    asdl = jnp.dot(xml, cl_ref[...], precision=_HI,
                   preferred_element_type=jnp.float32)
    asdu = jnp.dot(xmu, cu_ref[...], precision=_HI,
                   preferred_element_type=jnp.float32)
    asdl_ref[...] = asdl
    asdu_ref[...] = asdu
    _pre_maxupdate(i, n, asdl, asdu, maxes_ref, mxs_ref)


def _pre2(al, au, wxp, Wl, Wu, Ws, cl, cu):
    grid = N_EDGES // PRE_BLK
    outs = pl.pallas_call(
        _pre2_body,
        grid=(grid,),
        in_specs=[pl.BlockSpec((PRE_BLK, HID), lambda i: (i, 0))] * 3
        + [pl.BlockSpec((HID, HID), lambda i: (0, 0))] * 3
        + [pl.BlockSpec((HID, 16), lambda i: (0, 0))] * 2,
        out_specs=[
            pl.BlockSpec((PRE_BLK, HID), lambda i: (i, 0)),
            pl.BlockSpec((PRE_BLK, HID), lambda i: (i, 0)),
            pl.BlockSpec((PRE_BLK, HID), lambda i: (i, 0)),
            pl.BlockSpec((PRE_BLK, 16), lambda i: (i, 0)),
            pl.BlockSpec((PRE_BLK, 16), lambda i: (i, 0)),
            pl.BlockSpec((2, 16), lambda i: (0, 0)),
        ],
        out_shape=[
            jax.ShapeDtypeStruct((N_EDGES, HID), jnp.float32),
            jax.ShapeDtypeStruct((N_EDGES, HID), jnp.float32),
            jax.ShapeDtypeStruct((N_EDGES, HID), jnp.float32),
            jax.ShapeDtypeStruct((N_EDGES, 16), jnp.float32),
            jax.ShapeDtypeStruct((N_EDGES, 16), jnp.float32),
            jax.ShapeDtypeStruct((2, 16), jnp.float32),
        ],
        scratch_shapes=[pltpu.VMEM((8, 128), jnp.float32)],
    )(al, au, wxp, Wl, Wu, Ws, cl, cu)
    return outs


POOL_BLK = 8000


def _pool_mlp_body(al_ref, au_ref, wxp_ref, ap_ref, w0_ref, b0_ref, w1_ref,
                   b1_ref, out_ref, acc_ref):
    i = pl.program_id(0)
    n = pl.num_programs(0)
    xb = jax.nn.relu(al_ref[...] + au_ref[...] + wxp_ref[...])
    ap = ap_ref[...]
    s = jax.nn.sigmoid(jnp.sum(xb * ap, axis=1, keepdims=True))
    g = xb * s
    m = jnp.max(g, axis=0)

    @pl.when(i == 0)
    def _init():
        acc_ref[...] = jnp.full_like(acc_ref[...], -jnp.inf)

    acc_ref[0, :HID] = jnp.maximum(acc_ref[0, :HID], m)

    @pl.when(i == n - 1)
    def _final():
        y = acc_ref[0:1, :HID]
        h = jax.nn.relu(
            jnp.dot(y, w0_ref[...], preferred_element_type=jnp.float32)
            + b0_ref[...]
        )
        o = jax.nn.relu(jnp.sum(h * w1_ref[...], axis=1, keepdims=True) + b1_ref[...])
        out_ref[...] = o


def _pool_mlp(al, au, wxp, att_pool, W0, b0, W1, b1):
    grid = N_EDGES // POOL_BLK
    out = pl.pallas_call(
        _pool_mlp_body,
        grid=(grid,),
        in_specs=[pl.BlockSpec((POOL_BLK, HID), lambda i: (i, 0))] * 3 + [
            pl.BlockSpec((1, HID), lambda i: (0, 0)),
            pl.BlockSpec((HID, 64), lambda i: (0, 0)),
            pl.BlockSpec((1, 64), lambda i: (0, 0)),
            pl.BlockSpec((1, 64), lambda i: (0, 0)),
            pl.BlockSpec((1, 1), lambda i: (0, 0)),
        ],
        out_specs=pl.BlockSpec((1, 1), lambda i: (0, 0)),
        out_shape=jax.ShapeDtypeStruct((1, 1), jnp.float32),
        scratch_shapes=[pltpu.VMEM((8, 128), jnp.float32)],
    )(al, au, wxp, att_pool.reshape(1, HID), W0, b0.reshape(1, 64),
      W1.reshape(1, 64), b1.reshape(1, 1))
    return out.reshape(1)


def _cmat(a_src, a_dst):
    cmat = jnp.zeros((HID, 16), jnp.float32)
    for h in range(HEADS):
        cmat = cmat.at[h * OUT_CH:(h + 1) * OUT_CH, h].set(a_src[h])
        cmat = cmat.at[h * OUT_CH:(h + 1) * OUT_CH, 3 + h].set(a_dst[h])
    return cmat


def _mha_call(asd, xm, idx, mm):
    c = _leaky_v(jnp.max(mm[:HEADS] + mm[HEADS:2 * HEADS]))
    maxv = jnp.full((16,), c, jnp.float32)
    z2d = jnp.zeros((NBLK, AW), jnp.float32)
    z1d = jnp.zeros((4 * K,), jnp.float32)
    return _sc_mha(asd, xm, idx[1], idx[0], maxv, z2d, z1d)


def kernel(x_0, x_1, neighborhood_0_to_0, lower_neighborhood, upper_neighborhood,
           edge_indices, lift_att, Wl1, als1, ald1, Wu1, aus1, aud1, Ws1,
           Wl2, als2, ald2, Wu2, aus2, aud2, Ws2, att_pool, W0, b0, W1, b1):
    y1, y2 = _lift_proj(x_0, lift_att[:D0], lift_att[D0:])
    lifted = _sc_lift(y1, y2, neighborhood_0_to_0[0], neighborhood_0_to_0[1])

    xml, xmu, wx1, asdl, asdu, mx = _pre1(
        lifted, x_1, Wl1, Wu1, Ws1, _cmat(als1, ald1), _cmat(aus1, aud1))
    al = _mha_call(asdl, xml, lower_neighborhood, mx[0])
    au = _mha_call(asdu, xmu, upper_neighborhood, mx[1])

    xml2, xmu2, wx2, asdl2, asdu2, mx2 = _pre2(
        al, au, wx1, Wl2, Wu2, Ws2, _cmat(als2, ald2), _cmat(aus2, aud2))
    al2 = _mha_call(asdl2, xml2, lower_neighborhood, mx2[0])
    au2 = _mha_call(asdu2, xmu2, upper_neighborhood, mx2[1])

    return _pool_mlp(al2, au2, wx2, att_pool, W0, b0, W1, b1)


# speedup vs baseline: 53.9556x; 1.1503x over previous
import functools

import jax
import jax.numpy as jnp
from jax import lax
from jax.experimental import pallas as pl
from jax.experimental.pallas import tpu as pltpu
from jax.experimental.pallas import tpu_sc as plsc

N_NODES = 10000
N_EDGES = 160000
E_NB = 640000
D0 = 128
D1 = 16
HEADS = 3
OUT_CH = 32
HID = HEADS * OUT_CH

NRANGE = 16
RNG = N_EDGES // NRANGE
RPT = RNG // 16
NBLK = 25
EPT = E_NB // 16
ECH = 800
NCH = EPT // ECH
K = 128
SELCAP = ECH + K + 16
AW = 112

_mesh = plsc.VectorSubcoreMesh(core_axis_name="c", subcore_axis_name="s")


def _iota16():
    return jnp.arange(16, dtype=jnp.int32)


def _full16(v):
    return jnp.full((16,), v, dtype=jnp.int32)


def _leaky_v(x):
    return jnp.where(x >= 0, x, 0.2 * x)


@functools.partial(
    pl.kernel,
    out_type=jax.ShapeDtypeStruct((N_EDGES, HID), jnp.float32),
    mesh=_mesh,
    compiler_params=pltpu.CompilerParams(use_tc_tiling_on_sc=False,
                                         needs_layout_passes=False),
    scratch_types=dict(
        selt=pltpu.VMEM((SELCAP,), jnp.int32),
        selsrc=pltpu.VMEM((SELCAP,), jnp.int32),
        tch=pltpu.VMEM((ECH,), jnp.int32),
        sch=pltpu.VMEM((ECH,), jnp.int32),
        gsrc=pltpu.VMEM((K,), jnp.int32),
        gtgt=pltpu.VMEM((K,), jnp.int32),
        lidx=pltpu.VMEM((K,), jnp.int32),
        asd_s=pltpu.VMEM((K, 16), jnp.float32),
        asd_t=pltpu.VMEM((K, 16), jnp.float32),
        xmb=pltpu.VMEM((K, HID), jnp.float32),
        ebuf=pltpu.VMEM((4 * K,), jnp.float32),
        msgs=pltpu.VMEM((K, AW), jnp.float32),
        nbuf=pltpu.VMEM((NBLK, AW), jnp.float32),
        obuf=pltpu.VMEM((NBLK, HID), jnp.float32),
        mvbuf=pltpu.VMEM((16,), jnp.float32),
        sem_a=pltpu.SemaphoreType.DMA,
        sem_b=pltpu.SemaphoreType.DMA,
        sem_c=pltpu.SemaphoreType.DMA,
        acc=pltpu.VMEM_SHARED((RNG, AW), jnp.float32),
    ),
)
def _sc_mha(asd_hbm, xm_hbm, src_hbm, tgt_hbm, maxv_hbm, z2d_hbm, z1d_hbm,
            out_hbm,
            selt, selsrc, tch, sch, gsrc, gtgt, lidx,
            asd_s, asd_t, xmb, ebuf, msgs, nbuf, obuf, mvbuf,
            sem_a, sem_b, sem_c, acc):
    cid = lax.axis_index("c")
    tid = lax.axis_index("s")
    iota = _iota16()

    pltpu.sync_copy(z2d_hbm, nbuf)
    pltpu.sync_copy(z1d_hbm, ebuf)
    pltpu.sync_copy(maxv_hbm, mvbuf)
    cvec = mvbuf[...]

    zi = jnp.zeros((16,), jnp.int32)

    def _zs(i, _):
        selt[pl.ds(i * 16, 16)] = zi
        selsrc[pl.ds(i * 16, 16)] = zi
        return 0
    lax.fori_loop(0, SELCAP // 16, _zs, 0)

    def _flush_group(base, nsel, lo):
        def _mkidx(j, _c2):
            lt = selt[pl.ds(base + j * 16, 16)]
            ls = selsrc[pl.ds(base + j * 16, 16)]
            lidx[pl.ds(j * 16, 16)] = lt
            gtgt[pl.ds(j * 16, 16)] = lt + lo
            gsrc[pl.ds(j * 16, 16)] = ls
            return 0
        lax.fori_loop(0, K // 16, _mkidx, 0)

        ca = pltpu.async_copy(asd_hbm.at[gsrc], asd_s, sem_a)
        cb = pltpu.async_copy(asd_hbm.at[gtgt], asd_t, sem_b)
        cc = pltpu.async_copy(xm_hbm.at[gsrc], xmb, sem_c)
        ca.wait()
        cb.wait()
        cc.wait()

        def _egrp(j, _c3):
            rows = iota + j * 16
            valid = (base + rows) < nsel
            for h in range(HEADS):
                a1 = plsc.load_gather(asd_s, [rows, _full16(h)])
                a2 = plsc.load_gather(asd_t, [rows, _full16(3 + h)])
                e = jnp.exp(_leaky_v(a1 + a2) - cvec)
                e = jnp.where(valid, e, 0.0)
                ebuf[pl.ds(h * K + j * 16, 16)] = e
            return 0
        lax.fori_loop(0, K // 16, _egrp, 0)

        def _mrow(i, _c4):
            ev = plsc.load_gather(ebuf, [jnp.minimum(iota, 3) * K + i])
            msgs[i, pl.ds(HID, 16)] = ev
            for h in range(HEADS):
                eh = plsc.load_gather(ebuf, [_full16(h * K) + i])
                for b in range(2):
                    c0 = h * OUT_CH + b * 16
                    xv = xmb[i, pl.ds(c0, 16)]
                    msgs[i, pl.ds(c0, 16)] = xv * eh
            return 0
        lax.fori_loop(0, K, _mrow, 0)

        pltpu.sync_copy(msgs, acc.at[lidx], add=True)

    def _range_body(ri, _):
        p = cid + 2 * ri
        lo = p * RNG

        def _zacc(z, _c):
            pltpu.sync_copy(nbuf, acc.at[pl.ds(tid * RPT + z * NBLK, NBLK)])
            return 0
        lax.fori_loop(0, RPT // NBLK, _zacc, 0)
        plsc.subcore_barrier()

        def _chunk(ch, nsel):
            eoff = tid * EPT + ch * ECH
            da = pltpu.async_copy(tgt_hbm.at[pl.ds(eoff, ECH)], tch, sem_a)
            db = pltpu.async_copy(src_hbm.at[pl.ds(eoff, ECH)], sch, sem_b)
            da.wait()
            db.wait()

            def _vg(j, ns):
                t = tch[pl.ds(j * 16, 16)]
                s = sch[pl.ds(j * 16, 16)]
                inb = (t >= lo) & (t < lo + RNG)
                plsc.store_compressed(selt.at[pl.ds(ns, 16)], t - lo, mask=inb)
                plsc.store_compressed(selsrc.at[pl.ds(ns, 16)], s, mask=inb)
                cnt = jnp.max(plsc.all_reduce_population_count(inb))
                return ns + cnt
            nsel = lax.fori_loop(0, ECH // 16, _vg, nsel)

            ngr = nsel // K

            def _fl(g, _c):
                _flush_group(g * K, nsel, lo)
                return 0
            lax.fori_loop(0, ngr, _fl, 0)

            rem = nsel - ngr * K

            def _mv(j, _c):
                vt = selt[pl.ds(ngr * K + j * 16, 16)]
                vs = selsrc[pl.ds(ngr * K + j * 16, 16)]
                selt[pl.ds(j * 16, 16)] = vt
                selsrc[pl.ds(j * 16, 16)] = vs
                return 0
            lax.fori_loop(0, K // 16, _mv, 0)
            return rem
        nsel = lax.fori_loop(0, NCH, _chunk, jnp.int32(0))

        ngroups = (nsel + (K - 1)) // K

        def _flast(g, _c):
            _flush_group(g * K, nsel, lo)
            return 0
        lax.fori_loop(0, ngroups, _flast, 0)
        plsc.subcore_barrier()

        def _nblk(b, _c):
            row0 = tid * RPT + b * NBLK
            pltpu.sync_copy(acc.at[pl.ds(row0, NBLK)], nbuf)

            def _nrow(r, _c2):
                fr = _full16(r)
                for h in range(HEADS):
                    d = plsc.load_gather(nbuf, [fr, _full16(HID + h)])
                    d = d + 1e-16
                    for bb in range(2):
                        c0 = h * OUT_CH + bb * 16
                        u = nbuf[r, pl.ds(c0, 16)]
                        obuf[r, pl.ds(c0, 16)] = u / d
                return 0
            lax.fori_loop(0, NBLK, _nrow, 0)
            pltpu.sync_copy(obuf, out_hbm.at[pl.ds(lo + row0, NBLK)])
            return 0
        lax.fori_loop(0, RPT // NBLK, _nblk, 0)

        pltpu.sync_copy(z2d_hbm, nbuf)
        plsc.subcore_barrier()
        return 0

    lax.fori_loop(0, NRANGE // 2, _range_body, 0)


LCH = 200
LPT = N_EDGES // 32


@functools.partial(
    pl.kernel,
    out_type=jax.ShapeDtypeStruct((N_EDGES, D0), jnp.float32),
    mesh=_mesh,
    compiler_params=pltpu.CompilerParams(use_tc_tiling_on_sc=False,
                                         needs_layout_passes=False),
    scratch_types=dict(
        sidx=pltpu.VMEM((LCH,), jnp.int32),
        tidx=pltpu.VMEM((LCH,), jnp.int32),
        y1b=pltpu.VMEM((LCH, D0), jnp.float32),
        y2b=pltpu.VMEM((LCH, D0), jnp.float32),
        ob=pltpu.VMEM((LCH, D0), jnp.float32),
    ),
)
def _sc_lift(y1_hbm, y2_hbm, s_hbm, t_hbm, out_hbm, sidx, tidx, y1b, y2b, ob):
    cid = lax.axis_index("c")
    tid = lax.axis_index("s")
    wid = tid * 2 + cid
    base = wid * LPT

    def _chunk(ch, _):
        eoff = base + ch * LCH
        pltpu.sync_copy(s_hbm.at[pl.ds(eoff, LCH)], sidx)
        pltpu.sync_copy(t_hbm.at[pl.ds(eoff, LCH)], tidx)
        pltpu.sync_copy(y1_hbm.at[sidx], y1b)
        pltpu.sync_copy(y2_hbm.at[tidx], y2b)

        def _row(r, _c):
            for c8 in range(D0 // 16):
                v = y1b[r, pl.ds(c8 * 16, 16)] + y2b[r, pl.ds(c8 * 16, 16)]
                ob[r, pl.ds(c8 * 16, 16)] = jnp.maximum(v, 0.0)
            return 0
        lax.fori_loop(0, LCH, _row, 0)
        pltpu.sync_copy(ob, out_hbm.at[pl.ds(eoff, LCH)])
        return 0
    lax.fori_loop(0, LPT // LCH, _chunk, 0)


PRE_BLK = 4000
_HI = jax.lax.Precision.HIGHEST


def _lift_proj_body(x0_ref, a1_ref, a2_ref, y1_ref, y2_ref):
    x = x0_ref[...]
    y1_ref[...] = jnp.dot(x, a1_ref[...], preferred_element_type=jnp.float32)
    y2_ref[...] = jnp.dot(x, a2_ref[...], preferred_element_type=jnp.float32)


def _lift_proj(x_0, a1, a2):
    return pl.pallas_call(
        _lift_proj_body,
        grid=(5,),
        in_specs=[
            pl.BlockSpec((N_NODES // 5, D0), lambda i: (i, 0)),
            pl.BlockSpec((D0, D0), lambda i: (0, 0)),
            pl.BlockSpec((D0, D0), lambda i: (0, 0)),
        ],
        out_specs=[
            pl.BlockSpec((N_NODES // 5, D0), lambda i: (i, 0)),
            pl.BlockSpec((N_NODES // 5, D0), lambda i: (i, 0)),
        ],
        out_shape=[jax.ShapeDtypeStruct((N_NODES, D0), jnp.float32)] * 2,
    )(x_0, a1, a2)


def _pre_maxupdate(i, n, asdl, asdu, maxes_ref, mxs_ref):
    ml = jnp.max(asdl, axis=0).reshape(1, 16)
    mu = jnp.max(asdu, axis=0).reshape(1, 16)

    @pl.when(i == 0)
    def _init():
        mxs_ref[...] = jnp.full_like(mxs_ref[...], -jnp.inf)

    mxs_ref[0:1, :16] = jnp.maximum(mxs_ref[0:1, :16], ml)
    mxs_ref[1:2, :16] = jnp.maximum(mxs_ref[1:2, :16], mu)

    @pl.when(i == n - 1)
    def _final():
        maxes_ref[...] = mxs_ref[0:2, :16]


def _pre1_body(xa_ref, xb_ref, wla_ref, wlb_ref, wua_ref, wub_ref,
               wsa_ref, wsb_ref, cl_ref, cu_ref,
               xml_ref, xmu_ref, wx_ref, asdl_ref, asdu_ref, maxes_ref,
               mxs_ref):
    i = pl.program_id(0)
    n = pl.num_programs(0)
    xa = xa_ref[...]
    xb = xb_ref[...]
    xml = (jnp.dot(xa, wla_ref[...], preferred_element_type=jnp.float32)
           + jnp.dot(xb, wlb_ref[...], preferred_element_type=jnp.float32))
    xmu = (jnp.dot(xa, wua_ref[...], preferred_element_type=jnp.float32)
           + jnp.dot(xb, wub_ref[...], preferred_element_type=jnp.float32))
    wx = (jnp.dot(xa, wsa_ref[...], preferred_element_type=jnp.float32)
          + jnp.dot(xb, wsb_ref[...], preferred_element_type=jnp.float32))
    xml_ref[...] = xml
    xmu_ref[...] = xmu
    wx_ref[...] = wx * (1.0 + 1e-6)
    asdl = jnp.dot(xml, cl_ref[...], precision=_HI,
                   preferred_element_type=jnp.float32)
    asdu = jnp.dot(xmu, cu_ref[...], precision=_HI,
                   preferred_element_type=jnp.float32)
    asdl_ref[...] = asdl
    asdu_ref[...] = asdu
    _pre_maxupdate(i, n, asdl, asdu, maxes_ref, mxs_ref)


def _pre1(lifted, x_1, Wl, Wu, Ws, cl, cu):
    grid = N_EDGES // PRE_BLK
    outs = pl.pallas_call(
        _pre1_body,
        grid=(grid,),
        in_specs=[
            pl.BlockSpec((PRE_BLK, D0), lambda i: (i, 0)),
            pl.BlockSpec((PRE_BLK, D1), lambda i: (i, 0)),
        ] + [pl.BlockSpec((D0, HID), lambda i: (0, 0)),
             pl.BlockSpec((D1, HID), lambda i: (0, 0))] * 3
        + [pl.BlockSpec((HID, 16), lambda i: (0, 0))] * 2,
        out_specs=[
            pl.BlockSpec((PRE_BLK, HID), lambda i: (i, 0)),
            pl.BlockSpec((PRE_BLK, HID), lambda i: (i, 0)),
            pl.BlockSpec((PRE_BLK, HID), lambda i: (i, 0)),
            pl.BlockSpec((PRE_BLK, 16), lambda i: (i, 0)),
            pl.BlockSpec((PRE_BLK, 16), lambda i: (i, 0)),
            pl.BlockSpec((2, 16), lambda i: (0, 0)),
        ],
        out_shape=[
            jax.ShapeDtypeStruct((N_EDGES, HID), jnp.float32),
            jax.ShapeDtypeStruct((N_EDGES, HID), jnp.float32),
            jax.ShapeDtypeStruct((N_EDGES, HID), jnp.float32),
            jax.ShapeDtypeStruct((N_EDGES, 16), jnp.float32),
            jax.ShapeDtypeStruct((N_EDGES, 16), jnp.float32),
            jax.ShapeDtypeStruct((2, 16), jnp.float32),
        ],
        scratch_shapes=[pltpu.VMEM((8, 128), jnp.float32)],
    )(lifted, x_1, Wl[:D0], Wl[D0:], Wu[:D0], Wu[D0:], Ws[:D0], Ws[D0:],
      cl, cu)
    return outs


def _pre2_body(al_ref, au_ref, wxp_ref, wl_ref, wu_ref, ws_ref, cl_ref, cu_ref,
               xml_ref, xmu_ref, wx_ref, asdl_ref, asdu_ref, maxes_ref,
               mxs_ref):
    i = pl.program_id(0)
    n = pl.num_programs(0)
    x = jax.nn.relu(al_ref[...] + au_ref[...] + wxp_ref[...])
    xml = jnp.dot(x, wl_ref[...], preferred_element_type=jnp.float32)
    xmu = jnp.dot(x, wu_ref[...], preferred_element_type=jnp.float32)
    xml_ref[...] = xml
    xmu_ref[...] = xmu
    wx_ref[...] = jnp.dot(x, ws_ref[...],
                          preferred_element_type=jnp.float32) * (1.0 + 1e-6)
    asdl = jnp.dot(xml, cl_ref[...], precision=_HI,
                   preferred_element_type=jnp.float32)
    asdu = jnp.dot(xmu, cu_ref[...], precision=_HI,
                   preferred_element_type=jnp.float32)
    asdl_ref[...] = asdl
    asdu_ref[...] = asdu
    _pre_maxupdate(i, n, asdl, asdu, maxes_ref, mxs_ref)


def _pre2(al, au, wxp, Wl, Wu, Ws, cl, cu):
    grid = N_EDGES // PRE_BLK
    outs = pl.pallas_call(
        _pre2_body,
        grid=(grid,),
        in_specs=[pl.BlockSpec((PRE_BLK, HID), lambda i: (i, 0))] * 3
        + [pl.BlockSpec((HID, HID), lambda i: (0, 0))] * 3
        + [pl.BlockSpec((HID, 16), lambda i: (0, 0))] * 2,
        out_specs=[
            pl.BlockSpec((PRE_BLK, HID), lambda i: (i, 0)),
            pl.BlockSpec((PRE_BLK, HID), lambda i: (i, 0)),
            pl.BlockSpec((PRE_BLK, HID), lambda i: (i, 0)),
            pl.BlockSpec((PRE_BLK, 16), lambda i: (i, 0)),
            pl.BlockSpec((PRE_BLK, 16), lambda i: (i, 0)),
            pl.BlockSpec((2, 16), lambda i: (0, 0)),
        ],
        out_shape=[
            jax.ShapeDtypeStruct((N_EDGES, HID), jnp.float32),
            jax.ShapeDtypeStruct((N_EDGES, HID), jnp.float32),
            jax.ShapeDtypeStruct((N_EDGES, HID), jnp.float32),
            jax.ShapeDtypeStruct((N_EDGES, 16), jnp.float32),
            jax.ShapeDtypeStruct((N_EDGES, 16), jnp.float32),
            jax.ShapeDtypeStruct((2, 16), jnp.float32),
        ],
        scratch_shapes=[pltpu.VMEM((8, 128), jnp.float32)],
    )(al, au, wxp, Wl, Wu, Ws, cl, cu)
    return outs


POOL_BLK = 8000


def _pool_mlp_body(al_ref, au_ref, wxp_ref, ap_ref, w0_ref, b0_ref, w1_ref,
                   b1_ref, out_ref, acc_ref):
    i = pl.program_id(0)
    n = pl.num_programs(0)
    xb = jax.nn.relu(al_ref[...] + au_ref[...] + wxp_ref[...])
    ap = ap_ref[...]
    s = jax.nn.sigmoid(jnp.sum(xb * ap, axis=1, keepdims=True))
    g = xb * s
    m = jnp.max(g, axis=0)

    @pl.when(i == 0)
    def _init():
        acc_ref[...] = jnp.full_like(acc_ref[...], -jnp.inf)

    acc_ref[0, :HID] = jnp.maximum(acc_ref[0, :HID], m)

    @pl.when(i == n - 1)
    def _final():
        y = acc_ref[0:1, :HID]
        h = jax.nn.relu(
            jnp.dot(y, w0_ref[...], preferred_element_type=jnp.float32)
            + b0_ref[...]
        )
        o = jax.nn.relu(jnp.sum(h * w1_ref[...], axis=1, keepdims=True) + b1_ref[...])
        out_ref[...] = o


def _pool_mlp(al, au, wxp, att_pool, W0, b0, W1, b1):
    grid = N_EDGES // POOL_BLK
    out = pl.pallas_call(
        _pool_mlp_body,
        grid=(grid,),
        in_specs=[pl.BlockSpec((POOL_BLK, HID), lambda i: (i, 0))] * 3 + [
            pl.BlockSpec((1, HID), lambda i: (0, 0)),
            pl.BlockSpec((HID, 64), lambda i: (0, 0)),
            pl.BlockSpec((1, 64), lambda i: (0, 0)),
            pl.BlockSpec((1, 64), lambda i: (0, 0)),
            pl.BlockSpec((1, 1), lambda i: (0, 0)),
        ],
        out_specs=pl.BlockSpec((1, 1), lambda i: (0, 0)),
        out_shape=jax.ShapeDtypeStruct((1, 1), jnp.float32),
        scratch_shapes=[pltpu.VMEM((8, 128), jnp.float32)],
    )(al, au, wxp, att_pool.reshape(1, HID), W0, b0.reshape(1, 64),
      W1.reshape(1, 64), b1.reshape(1, 1))
    return out.reshape(1)


def _cmat(a_src, a_dst):
    cmat = jnp.zeros((HID, 16), jnp.float32)
    for h in range(HEADS):
        cmat = cmat.at[h * OUT_CH:(h + 1) * OUT_CH, h].set(a_src[h])
        cmat = cmat.at[h * OUT_CH:(h + 1) * OUT_CH, 3 + h].set(a_dst[h])
    return cmat


def _mha_call(asd, xm, idx, mm):
    c = _leaky_v(jnp.max(mm[:HEADS] + mm[HEADS:2 * HEADS]))
    maxv = jnp.full((16,), c, jnp.float32)
    z2d = jnp.zeros((NBLK, AW), jnp.float32)
    z1d = jnp.zeros((4 * K,), jnp.float32)
    return _sc_mha(asd, xm, idx[1], idx[0], maxv, z2d, z1d)


def kernel(x_0, x_1, neighborhood_0_to_0, lower_neighborhood, upper_neighborhood,
           edge_indices, lift_att, Wl1, als1, ald1, Wu1, aus1, aud1, Ws1,
           Wl2, als2, ald2, Wu2, aus2, aud2, Ws2, att_pool, W0, b0, W1, b1):
    y1, y2 = _lift_proj(x_0, lift_att[:D0], lift_att[D0:])
    lifted = _sc_lift(y1, y2, neighborhood_0_to_0[0], neighborhood_0_to_0[1])

    xml, xmu, wx1, asdl, asdu, mx = _pre1(
        lifted, x_1, Wl1, Wu1, Ws1, _cmat(als1, ald1), _cmat(aus1, aud1))
    al = _mha_call(asdl, xml, lower_neighborhood, mx[0])
    au = _mha_call(asdu, xmu, upper_neighborhood, mx[1])

    xml2, xmu2, wx2, asdl2, asdu2, mx2 = _pre2(
        al, au, wx1, Wl2, Wu2, Ws2, _cmat(als2, ald2), _cmat(aus2, aud2))
    al2 = _mha_call(asdl2, xml2, lower_neighborhood, mx2[0])
    au2 = _mha_call(asdu2, xmu2, upper_neighborhood, mx2[1])

    return _pool_mlp(al2, au2, wx2, att_pool, W0, b0, W1, b1)


# speedup vs baseline: 64.2826x; 1.1914x over previous
import functools

import jax
import jax.numpy as jnp
from jax import lax
from jax.experimental import pallas as pl
from jax.experimental.pallas import tpu as pltpu
from jax.experimental.pallas import tpu_sc as plsc

N_NODES = 10000
N_EDGES = 160000
E_NB = 640000
D0 = 128
D1 = 16
HEADS = 3
OUT_CH = 32
HID = HEADS * OUT_CH

NRANGE = 16
RNG = N_EDGES // NRANGE
RPT = RNG // 16
NBLK = 25
EPT = E_NB // 16
ECH = 4000
NCH = EPT // ECH
K = 128
SELCAP = ECH + K + 16
AW = 112

_mesh = plsc.VectorSubcoreMesh(core_axis_name="c", subcore_axis_name="s")


def _iota16():
    return jnp.arange(16, dtype=jnp.int32)


def _full16(v):
    return jnp.full((16,), v, dtype=jnp.int32)


def _leaky_v(x):
    return jnp.where(x >= 0, x, 0.2 * x)


@functools.partial(
    pl.kernel,
    out_type=jax.ShapeDtypeStruct((N_EDGES, HID), jnp.float32),
    mesh=_mesh,
    compiler_params=pltpu.CompilerParams(use_tc_tiling_on_sc=False,
                                         needs_layout_passes=False),
    scratch_types=dict(
        selt=pltpu.VMEM((SELCAP,), jnp.int32),
        selsrc=pltpu.VMEM((SELCAP,), jnp.int32),
        tch=pltpu.VMEM((ECH,), jnp.int32),
        sch=pltpu.VMEM((ECH,), jnp.int32),
        gsrc=pltpu.VMEM((K,), jnp.int32),
        gtgt=pltpu.VMEM((K,), jnp.int32),
        lidx=pltpu.VMEM((K,), jnp.int32),
        asd_s=pltpu.VMEM((K, 16), jnp.float32),
        asd_t=pltpu.VMEM((K, 16), jnp.float32),
        xmb=pltpu.VMEM((K, HID), jnp.float32),
        ebuf=pltpu.VMEM((4 * K,), jnp.float32),
        msgs=pltpu.VMEM((K, AW), jnp.float32),
        nbuf=pltpu.VMEM((NBLK, AW), jnp.float32),
        obuf=pltpu.VMEM((NBLK, HID), jnp.float32),
        mvbuf=pltpu.VMEM((16,), jnp.float32),
        sem_a=pltpu.SemaphoreType.DMA,
        sem_b=pltpu.SemaphoreType.DMA,
        sem_c=pltpu.SemaphoreType.DMA,
        acc=pltpu.VMEM_SHARED((RNG, AW), jnp.float32),
    ),
)
def _sc_mha(asd_hbm, xm_hbm, src_hbm, tgt_hbm, maxv_hbm, z2d_hbm, z1d_hbm,
            out_hbm,
            selt, selsrc, tch, sch, gsrc, gtgt, lidx,
            asd_s, asd_t, xmb, ebuf, msgs, nbuf, obuf, mvbuf,
            sem_a, sem_b, sem_c, acc):
    cid = lax.axis_index("c")
    tid = lax.axis_index("s")
    iota = _iota16()

    pltpu.sync_copy(z2d_hbm, nbuf)
    pltpu.sync_copy(z1d_hbm, ebuf)
    pltpu.sync_copy(maxv_hbm, mvbuf)
    cvec = mvbuf[...]

    zi = jnp.zeros((16,), jnp.int32)

    def _zs(i, _):
        selt[pl.ds(i * 16, 16)] = zi
        selsrc[pl.ds(i * 16, 16)] = zi
        return 0
    lax.fori_loop(0, SELCAP // 16, _zs, 0)

    def _flush_group(base, nsel, lo):
        def _mkidx(j, _c2):
            lt = selt[pl.ds(base + j * 16, 16)]
            ls = selsrc[pl.ds(base + j * 16, 16)]
            lidx[pl.ds(j * 16, 16)] = lt
            gtgt[pl.ds(j * 16, 16)] = lt + lo
            gsrc[pl.ds(j * 16, 16)] = ls
            return 0
        lax.fori_loop(0, K // 16, _mkidx, 0)

        ca = pltpu.async_copy(asd_hbm.at[gsrc], asd_s, sem_a)
        cb = pltpu.async_copy(asd_hbm.at[gtgt], asd_t, sem_b)
        cc = pltpu.async_copy(xm_hbm.at[gsrc], xmb, sem_c)
        ca.wait()
        cb.wait()
        cc.wait()

        def _egrp(j, _c3):
            rows = iota + j * 16
            valid = (base + rows) < nsel
            for h in range(HEADS):
                a1 = plsc.load_gather(asd_s, [rows, _full16(h)])
                a2 = plsc.load_gather(asd_t, [rows, _full16(3 + h)])
                e = jnp.exp(_leaky_v(a1 + a2) - cvec)
                e = jnp.where(valid, e, 0.0)
                ebuf[pl.ds(h * K + j * 16, 16)] = e
            return 0
        lax.fori_loop(0, K // 16, _egrp, 0)

        def _mrow(i, _c4):
            ev = plsc.load_gather(ebuf, [jnp.minimum(iota, 3) * K + i])
            msgs[i, pl.ds(HID, 16)] = ev
            for h in range(HEADS):
                eh = plsc.load_gather(ebuf, [_full16(h * K) + i])
                for b in range(2):
                    c0 = h * OUT_CH + b * 16
                    xv = xmb[i, pl.ds(c0, 16)]
                    msgs[i, pl.ds(c0, 16)] = xv * eh
            return 0
        lax.fori_loop(0, K, _mrow, 0)

        pltpu.sync_copy(msgs, acc.at[lidx], add=True)

    def _range_body(ri, _):
        p = cid + 2 * ri
        lo = p * RNG

        def _zacc(z, _c):
            pltpu.sync_copy(nbuf, acc.at[pl.ds(tid * RPT + z * NBLK, NBLK)])
            return 0
        lax.fori_loop(0, RPT // NBLK, _zacc, 0)
        plsc.subcore_barrier()

        def _chunk(ch, nsel):
            eoff = tid * EPT + ch * ECH
            da = pltpu.async_copy(tgt_hbm.at[pl.ds(eoff, ECH)], tch, sem_a)
            db = pltpu.async_copy(src_hbm.at[pl.ds(eoff, ECH)], sch, sem_b)
            da.wait()
            db.wait()

            def _vg(j, ns):
                t = tch[pl.ds(j * 16, 16)]
                s = sch[pl.ds(j * 16, 16)]
                inb = (t >= lo) & (t < lo + RNG)
                plsc.store_compressed(selt.at[pl.ds(ns, 16)], t - lo, mask=inb)
                plsc.store_compressed(selsrc.at[pl.ds(ns, 16)], s, mask=inb)
                cnt = jnp.max(plsc.all_reduce_population_count(inb))
                return ns + cnt
            nsel = lax.fori_loop(0, ECH // 16, _vg, nsel)

            ngr = nsel // K

            def _fl(g, _c):
                _flush_group(g * K, nsel, lo)
                return 0
            lax.fori_loop(0, ngr, _fl, 0)

            rem = nsel - ngr * K

            def _mv(j, _c):
                vt = selt[pl.ds(ngr * K + j * 16, 16)]
                vs = selsrc[pl.ds(ngr * K + j * 16, 16)]
                selt[pl.ds(j * 16, 16)] = vt
                selsrc[pl.ds(j * 16, 16)] = vs
                return 0
            lax.fori_loop(0, K // 16, _mv, 0)
            return rem
        nsel = lax.fori_loop(0, NCH, _chunk, jnp.int32(0))

        ngroups = (nsel + (K - 1)) // K

        def _flast(g, _c):
            _flush_group(g * K, nsel, lo)
            return 0
        lax.fori_loop(0, ngroups, _flast, 0)
        plsc.subcore_barrier()

        def _nblk(b, _c):
            row0 = tid * RPT + b * NBLK
            pltpu.sync_copy(acc.at[pl.ds(row0, NBLK)], nbuf)

            def _nrow(r, _c2):
                fr = _full16(r)
                for h in range(HEADS):
                    d = plsc.load_gather(nbuf, [fr, _full16(HID + h)])
                    d = d + 1e-16
                    for bb in range(2):
                        c0 = h * OUT_CH + bb * 16
                        u = nbuf[r, pl.ds(c0, 16)]
                        obuf[r, pl.ds(c0, 16)] = u / d
                return 0
            lax.fori_loop(0, NBLK, _nrow, 0)
            pltpu.sync_copy(obuf, out_hbm.at[pl.ds(lo + row0, NBLK)])
            return 0
        lax.fori_loop(0, RPT // NBLK, _nblk, 0)

        pltpu.sync_copy(z2d_hbm, nbuf)
        plsc.subcore_barrier()
        return 0

    lax.fori_loop(0, NRANGE // 2, _range_body, 0)


LCH = 200
LPT = N_EDGES // 32


@functools.partial(
    pl.kernel,
    out_type=jax.ShapeDtypeStruct((N_EDGES, D0), jnp.float32),
    mesh=_mesh,
    compiler_params=pltpu.CompilerParams(use_tc_tiling_on_sc=False,
                                         needs_layout_passes=False),
    scratch_types=dict(
        sidx=pltpu.VMEM((LCH,), jnp.int32),
        tidx=pltpu.VMEM((LCH,), jnp.int32),
        y1b=pltpu.VMEM((LCH, D0), jnp.float32),
        y2b=pltpu.VMEM((LCH, D0), jnp.float32),
        ob=pltpu.VMEM((LCH, D0), jnp.float32),
    ),
)
def _sc_lift(y1_hbm, y2_hbm, s_hbm, t_hbm, out_hbm, sidx, tidx, y1b, y2b, ob):
    cid = lax.axis_index("c")
    tid = lax.axis_index("s")
    wid = tid * 2 + cid
    base = wid * LPT

    def _chunk(ch, _):
        eoff = base + ch * LCH
        pltpu.sync_copy(s_hbm.at[pl.ds(eoff, LCH)], sidx)
        pltpu.sync_copy(t_hbm.at[pl.ds(eoff, LCH)], tidx)
        pltpu.sync_copy(y1_hbm.at[sidx], y1b)
        pltpu.sync_copy(y2_hbm.at[tidx], y2b)

        def _row(r, _c):
            for c8 in range(D0 // 16):
                v = y1b[r, pl.ds(c8 * 16, 16)] + y2b[r, pl.ds(c8 * 16, 16)]
                ob[r, pl.ds(c8 * 16, 16)] = jnp.maximum(v, 0.0)
            return 0
        lax.fori_loop(0, LCH, _row, 0)
        pltpu.sync_copy(ob, out_hbm.at[pl.ds(eoff, LCH)])
        return 0
    lax.fori_loop(0, LPT // LCH, _chunk, 0)


PRE_BLK = 4000
_HI = jax.lax.Precision.HIGHEST


def _lift_proj_body(x0_ref, a1_ref, a2_ref, y1_ref, y2_ref):
    x = x0_ref[...]
    y1_ref[...] = jnp.dot(x, a1_ref[...], preferred_element_type=jnp.float32)
    y2_ref[...] = jnp.dot(x, a2_ref[...], preferred_element_type=jnp.float32)


def _lift_proj(x_0, a1, a2):
    return pl.pallas_call(
        _lift_proj_body,
        grid=(5,),
        in_specs=[
            pl.BlockSpec((N_NODES // 5, D0), lambda i: (i, 0)),
            pl.BlockSpec((D0, D0), lambda i: (0, 0)),
            pl.BlockSpec((D0, D0), lambda i: (0, 0)),
        ],
        out_specs=[
            pl.BlockSpec((N_NODES // 5, D0), lambda i: (i, 0)),
            pl.BlockSpec((N_NODES // 5, D0), lambda i: (i, 0)),
        ],
        out_shape=[jax.ShapeDtypeStruct((N_NODES, D0), jnp.float32)] * 2,
    )(x_0, a1, a2)


def _pre_maxupdate(i, n, asdl, asdu, maxes_ref, mxs_ref):
    ml = jnp.max(asdl, axis=0).reshape(1, 16)
    mu = jnp.max(asdu, axis=0).reshape(1, 16)

    @pl.when(i == 0)
    def _init():
        mxs_ref[...] = jnp.full_like(mxs_ref[...], -jnp.inf)

    mxs_ref[0:1, :16] = jnp.maximum(mxs_ref[0:1, :16], ml)
    mxs_ref[1:2, :16] = jnp.maximum(mxs_ref[1:2, :16], mu)

    @pl.when(i == n - 1)
    def _final():
        maxes_ref[...] = mxs_ref[0:2, :16]


def _pre1_body(xa_ref, xb_ref, wla_ref, wlb_ref, wua_ref, wub_ref,
               wsa_ref, wsb_ref, cl_ref, cu_ref,
               xml_ref, xmu_ref, wx_ref, asdl_ref, asdu_ref, maxes_ref,
               mxs_ref):
    i = pl.program_id(0)
    n = pl.num_programs(0)
    xa = xa_ref[...]
    xb = xb_ref[...]
    xml = (jnp.dot(xa, wla_ref[...], preferred_element_type=jnp.float32)
           + jnp.dot(xb, wlb_ref[...], preferred_element_type=jnp.float32))
    xmu = (jnp.dot(xa, wua_ref[...], preferred_element_type=jnp.float32)
           + jnp.dot(xb, wub_ref[...], preferred_element_type=jnp.float32))
    wx = (jnp.dot(xa, wsa_ref[...], preferred_element_type=jnp.float32)
          + jnp.dot(xb, wsb_ref[...], preferred_element_type=jnp.float32))
    xml_ref[...] = xml
    xmu_ref[...] = xmu
    wx_ref[...] = wx * (1.0 + 1e-6)
    asdl = jnp.dot(xml, cl_ref[...], precision=_HI,
                   preferred_element_type=jnp.float32)
    asdu = jnp.dot(xmu, cu_ref[...], precision=_HI,
                   preferred_element_type=jnp.float32)
    asdl_ref[...] = asdl
    asdu_ref[...] = asdu
    _pre_maxupdate(i, n, asdl, asdu, maxes_ref, mxs_ref)


def _pre1(lifted, x_1, Wl, Wu, Ws, cl, cu):
    grid = N_EDGES // PRE_BLK
    outs = pl.pallas_call(
        _pre1_body,
        grid=(grid,),
        in_specs=[
            pl.BlockSpec((PRE_BLK, D0), lambda i: (i, 0)),
            pl.BlockSpec((PRE_BLK, D1), lambda i: (i, 0)),
        ] + [pl.BlockSpec((D0, HID), lambda i: (0, 0)),
             pl.BlockSpec((D1, HID), lambda i: (0, 0))] * 3
        + [pl.BlockSpec((HID, 16), lambda i: (0, 0))] * 2,
        out_specs=[
            pl.BlockSpec((PRE_BLK, HID), lambda i: (i, 0)),
            pl.BlockSpec((PRE_BLK, HID), lambda i: (i, 0)),
            pl.BlockSpec((PRE_BLK, HID), lambda i: (i, 0)),
            pl.BlockSpec((PRE_BLK, 16), lambda i: (i, 0)),
            pl.BlockSpec((PRE_BLK, 16), lambda i: (i, 0)),
            pl.BlockSpec((2, 16), lambda i: (0, 0)),
        ],
        out_shape=[
            jax.ShapeDtypeStruct((N_EDGES, HID), jnp.float32),
            jax.ShapeDtypeStruct((N_EDGES, HID), jnp.float32),
            jax.ShapeDtypeStruct((N_EDGES, HID), jnp.float32),
            jax.ShapeDtypeStruct((N_EDGES, 16), jnp.float32),
            jax.ShapeDtypeStruct((N_EDGES, 16), jnp.float32),
            jax.ShapeDtypeStruct((2, 16), jnp.float32),
        ],
        scratch_shapes=[pltpu.VMEM((8, 128), jnp.float32)],
    )(lifted, x_1, Wl[:D0], Wl[D0:], Wu[:D0], Wu[D0:], Ws[:D0], Ws[D0:],
      cl, cu)
    return outs


def _pre2_body(al_ref, au_ref, wxp_ref, wl_ref, wu_ref, ws_ref, cl_ref, cu_ref,
               xml_ref, xmu_ref, wx_ref, asdl_ref, asdu_ref, maxes_ref,
               mxs_ref):
    i = pl.program_id(0)
    n = pl.num_programs(0)
    x = jax.nn.relu(al_ref[...] + au_ref[...] + wxp_ref[...])
    xml = jnp.dot(x, wl_ref[...], preferred_element_type=jnp.float32)
    xmu = jnp.dot(x, wu_ref[...], preferred_element_type=jnp.float32)
    xml_ref[...] = xml
    xmu_ref[...] = xmu
    wx_ref[...] = jnp.dot(x, ws_ref[...],
                          preferred_element_type=jnp.float32) * (1.0 + 1e-6)
    asdl = jnp.dot(xml, cl_ref[...], precision=_HI,
                   preferred_element_type=jnp.float32)
    asdu = jnp.dot(xmu, cu_ref[...], precision=_HI,
                   preferred_element_type=jnp.float32)
    asdl_ref[...] = asdl
    asdu_ref[...] = asdu
    _pre_maxupdate(i, n, asdl, asdu, maxes_ref, mxs_ref)


def _pre2(al, au, wxp, Wl, Wu, Ws, cl, cu):
    grid = N_EDGES // PRE_BLK
    outs = pl.pallas_call(
        _pre2_body,
        grid=(grid,),
        in_specs=[pl.BlockSpec((PRE_BLK, HID), lambda i: (i, 0))] * 3
        + [pl.BlockSpec((HID, HID), lambda i: (0, 0))] * 3
        + [pl.BlockSpec((HID, 16), lambda i: (0, 0))] * 2,
        out_specs=[
            pl.BlockSpec((PRE_BLK, HID), lambda i: (i, 0)),
            pl.BlockSpec((PRE_BLK, HID), lambda i: (i, 0)),
            pl.BlockSpec((PRE_BLK, HID), lambda i: (i, 0)),
            pl.BlockSpec((PRE_BLK, 16), lambda i: (i, 0)),
            pl.BlockSpec((PRE_BLK, 16), lambda i: (i, 0)),
            pl.BlockSpec((2, 16), lambda i: (0, 0)),
        ],
        out_shape=[
            jax.ShapeDtypeStruct((N_EDGES, HID), jnp.float32),
            jax.ShapeDtypeStruct((N_EDGES, HID), jnp.float32),
            jax.ShapeDtypeStruct((N_EDGES, HID), jnp.float32),
            jax.ShapeDtypeStruct((N_EDGES, 16), jnp.float32),
            jax.ShapeDtypeStruct((N_EDGES, 16), jnp.float32),
            jax.ShapeDtypeStruct((2, 16), jnp.float32),
        ],
        scratch_shapes=[pltpu.VMEM((8, 128), jnp.float32)],
    )(al, au, wxp, Wl, Wu, Ws, cl, cu)
    return outs


POOL_BLK = 8000


def _pool_mlp_body(al_ref, au_ref, wxp_ref, ap_ref, w0_ref, b0_ref, w1_ref,
                   b1_ref, out_ref, acc_ref):
    i = pl.program_id(0)
    n = pl.num_programs(0)
    xb = jax.nn.relu(al_ref[...] + au_ref[...] + wxp_ref[...])
    ap = ap_ref[...]
    s = jax.nn.sigmoid(jnp.sum(xb * ap, axis=1, keepdims=True))
    g = xb * s
    m = jnp.max(g, axis=0)

    @pl.when(i == 0)
    def _init():
        acc_ref[...] = jnp.full_like(acc_ref[...], -jnp.inf)

    acc_ref[0, :HID] = jnp.maximum(acc_ref[0, :HID], m)

    @pl.when(i == n - 1)
    def _final():
        y = acc_ref[0:1, :HID]
        h = jax.nn.relu(
            jnp.dot(y, w0_ref[...], preferred_element_type=jnp.float32)
            + b0_ref[...]
        )
        o = jax.nn.relu(jnp.sum(h * w1_ref[...], axis=1, keepdims=True) + b1_ref[...])
        out_ref[...] = o


def _pool_mlp(al, au, wxp, att_pool, W0, b0, W1, b1):
    grid = N_EDGES // POOL_BLK
    out = pl.pallas_call(
        _pool_mlp_body,
        grid=(grid,),
        in_specs=[pl.BlockSpec((POOL_BLK, HID), lambda i: (i, 0))] * 3 + [
            pl.BlockSpec((1, HID), lambda i: (0, 0)),
            pl.BlockSpec((HID, 64), lambda i: (0, 0)),
            pl.BlockSpec((1, 64), lambda i: (0, 0)),
            pl.BlockSpec((1, 64), lambda i: (0, 0)),
            pl.BlockSpec((1, 1), lambda i: (0, 0)),
        ],
        out_specs=pl.BlockSpec((1, 1), lambda i: (0, 0)),
        out_shape=jax.ShapeDtypeStruct((1, 1), jnp.float32),
        scratch_shapes=[pltpu.VMEM((8, 128), jnp.float32)],
    )(al, au, wxp, att_pool.reshape(1, HID), W0, b0.reshape(1, 64),
      W1.reshape(1, 64), b1.reshape(1, 1))
    return out.reshape(1)


def _cmat(a_src, a_dst):
    cmat = jnp.zeros((HID, 16), jnp.float32)
    for h in range(HEADS):
        cmat = cmat.at[h * OUT_CH:(h + 1) * OUT_CH, h].set(a_src[h])
        cmat = cmat.at[h * OUT_CH:(h + 1) * OUT_CH, 3 + h].set(a_dst[h])
    return cmat


def _mha_call(asd, xm, idx, mm):
    c = _leaky_v(jnp.max(mm[:HEADS] + mm[HEADS:2 * HEADS]))
    maxv = jnp.full((16,), c, jnp.float32)
    z2d = jnp.zeros((NBLK, AW), jnp.float32)
    z1d = jnp.zeros((4 * K,), jnp.float32)
    return _sc_mha(asd, xm, idx[1], idx[0], maxv, z2d, z1d)


def kernel(x_0, x_1, neighborhood_0_to_0, lower_neighborhood, upper_neighborhood,
           edge_indices, lift_att, Wl1, als1, ald1, Wu1, aus1, aud1, Ws1,
           Wl2, als2, ald2, Wu2, aus2, aud2, Ws2, att_pool, W0, b0, W1, b1):
    y1, y2 = _lift_proj(x_0, lift_att[:D0], lift_att[D0:])
    lifted = _sc_lift(y1, y2, neighborhood_0_to_0[0], neighborhood_0_to_0[1])

    xml, xmu, wx1, asdl, asdu, mx = _pre1(
        lifted, x_1, Wl1, Wu1, Ws1, _cmat(als1, ald1), _cmat(aus1, aud1))
    al = _mha_call(asdl, xml, lower_neighborhood, mx[0])
    au = _mha_call(asdu, xmu, upper_neighborhood, mx[1])

    xml2, xmu2, wx2, asdl2, asdu2, mx2 = _pre2(
        al, au, wx1, Wl2, Wu2, Ws2, _cmat(als2, ald2), _cmat(aus2, aud2))
    al2 = _mha_call(asdl2, xml2, lower_neighborhood, mx2[0])
    au2 = _mha_call(asdu2, xmu2, upper_neighborhood, mx2[1])

    return _pool_mlp(al2, au2, wx2, att_pool, W0, b0, W1, b1)
